# baseline (speedup 1.0000x reference)
import functools

import jax
import jax.numpy as jnp
from jax.experimental import pallas as pl
from jax.experimental.pallas import tpu as pltpu
from jax.experimental.pallas import tpu_sc as plsc

C = 512
H = 50
W = 50
N = 512
POOL = 7
SCALE = 1.0 / 16.0
NUM_CLASSES = 21
FC = 1024
NBINS = POOL * POOL
NIDX = 4 * N * NBINS
KCH = POOL * C
GATHER_WIN = 128


def _bilinear_1d(lo, hi):
    lo = lo * SCALE
    hi = hi * SCALE
    ext = jnp.maximum(hi - lo, 1.0)
    g = (jax.lax.broadcasted_iota(jnp.int32, (1, POOL), 1).astype(jnp.float32)
         + 0.5) / POOL
    pos = lo + ext * g
    p0f = jnp.floor(pos)
    frac = pos - p0f
    return p0f, frac


def _corner_idx(p0f, limit):
    i0 = jnp.clip(p0f.astype(jnp.int32), 0, limit - 1)
    i1 = jnp.clip(i0 + 1, 0, limit - 1)
    return i0, i1


def _idx_body(prop_ref, idx_ref):
    p = prop_ref[...]
    x0f, _ = _bilinear_1d(p[:, 0:1], p[:, 2:3])
    y0f, _ = _bilinear_1d(p[:, 1:2], p[:, 3:4])
    x0, x1 = _corner_idx(x0f, W)
    y0, y1 = _corner_idx(y0f, H)
    x0t, x1t = jnp.transpose(x0), jnp.transpose(x1)
    y0t, y1t = jnp.transpose(y0), jnp.transpose(y1)
    for j, (yy, xx) in enumerate(((y0t, x0t), (y0t, x1t), (y1t, x0t), (y1t, x1t))):
        idx_ref[:, :, j, :] = yy[:, None, :] * W + xx[None, :, :]


def _compute_indices(proposals):
    return pl.pallas_call(
        _idx_body,
        out_shape=jax.ShapeDtypeStruct((POOL, POOL, 4, N), jnp.int32),
    )(proposals)


def _gather_rows(table, idx):
    mesh = plsc.VectorSubcoreMesh(core_axis_name="c", subcore_axis_name="s")
    D = table.shape[1]
    nidx = idx.shape[0] * idx.shape[1]

    @functools.partial(
        pl.kernel,
        out_type=jax.ShapeDtypeStruct((nidx, D), table.dtype),
        mesh=mesh,
    )
    def k(x_hbm, i_hbm, o_hbm):
        def body(i_vmem, o_vmem):
            pltpu.sync_copy(x_hbm.at[i_vmem.at[0]], o_vmem)

        blocks_per_row = idx.shape[1] // GATHER_WIN
        pltpu.emit_pipeline(
            body,
            grid=(nidx // GATHER_WIN,),
            in_specs=[pl.BlockSpec(
                (1, GATHER_WIN),
                lambda i: (i // blocks_per_row, i % blocks_per_row))],
            out_specs=[pl.BlockSpec((GATHER_WIN, D), lambda i: (i, 0))],
            core_axis_name=("c", "s"),
            dimension_semantics=(pltpu.PARALLEL,),
        )(i_hbm, o_hbm)

    return k(table, idx)


def _pack_body(f_ref, out_ref):
    ii = jax.lax.broadcasted_iota(jnp.int32, (C, C), 0)
    jj = jax.lax.broadcasted_iota(jnp.int32, (C, C), 1)
    ident = (ii == jj).astype(jnp.bfloat16)
    t = jax.lax.dot_general(
        f_ref[...].astype(jnp.bfloat16), ident, (((0,), (0,)), ((), ())),
        preferred_element_type=jnp.float32)
    lo = jax.lax.bitcast_convert_type(t[:, :C // 2], jnp.int32)
    hi = jax.lax.bitcast_convert_type(t[:, C // 2:], jnp.int32)
    himask = jnp.int32(-65536)
    out_ref[...] = (hi & himask) | ((lo >> 16) & jnp.int32(0xFFFF))


def _pack_table(fmap2d):
    return pl.pallas_call(
        _pack_body,
        grid=(pl.cdiv(H * W, 128),),
        in_specs=[pl.BlockSpec((C, 128), lambda i: (0, i))],
        out_specs=pl.BlockSpec((128, C // 2), lambda i: (i, 0)),
        out_shape=jax.ShapeDtypeStruct((H * W, C // 2), jnp.int32),
    )(fmap2d)


NB = 64


def _combine_body(prop_ref, v_ref, out_ref):
    p = prop_ref[...]
    _, fx = _bilinear_1d(p[:, 0:1], p[:, 2:3])
    _, fy = _bilinear_1d(p[:, 1:2], p[:, 3:4])
    hx, lx = 1.0 - fx, fx
    hy, ly = 1.0 - fy, fy
    himask = jnp.int32(-65536)
    for b in range(NBINS):
        py, px = b // POOL, b % POOL
        wy0 = hy[:, py:py + 1]
        wy1 = ly[:, py:py + 1]
        wx0 = hx[:, px:px + 1]
        wx1 = lx[:, px:px + 1]
        ws = (wy0 * wx0, wy0 * wx1, wy1 * wx0, wy1 * wx1)
        acc_lo = jnp.zeros((NB, C // 2), jnp.float32)
        acc_hi = jnp.zeros((NB, C // 2), jnp.float32)
        for j in range(4):
            slab = v_ref[b, j, :, :]
            lo = jax.lax.bitcast_convert_type(slab << 16, jnp.float32)
            hi = jax.lax.bitcast_convert_type(slab & himask, jnp.float32)
            acc_lo += ws[j] * lo
            acc_hi += ws[j] * hi
        out_ref[b, :, 0:C // 2] = acc_lo.astype(jnp.bfloat16)
        out_ref[b, :, C // 2:C] = acc_hi.astype(jnp.bfloat16)


def _combine(proposals, v4):
    return pl.pallas_call(
        _combine_body,
        grid=(N // NB,),
        in_specs=[
            pl.BlockSpec((NB, 4), lambda i: (i, 0)),
            pl.BlockSpec((NBINS, 4, NB, C // 2), lambda i: (0, 0, i, 0)),
        ],
        out_specs=pl.BlockSpec((NBINS, NB, C), lambda i: (0, i, 0)),
        out_shape=jax.ShapeDtypeStruct((NBINS, N, C), jnp.bfloat16),
    )(proposals, v4)


NT = 128


def _xpose_body(in_ref, out_ref):
    ii = jax.lax.broadcasted_iota(jnp.int32, (NT, NT), 0)
    jj = jax.lax.broadcasted_iota(jnp.int32, (NT, NT), 1)
    ident = (ii == jj).astype(jnp.bfloat16)
    for b in range(NBINS):
        t = jax.lax.dot_general(
            in_ref[b], ident, (((0,), (0,)), ((), ())),
            preferred_element_type=jnp.float32)
        out_ref[:, b, :] = t.astype(jnp.bfloat16)


def _xpose(pooled):
    return pl.pallas_call(
        _xpose_body,
        grid=(N // NT,),
        in_specs=[pl.BlockSpec((NBINS, NT, C), lambda i: (0, i, 0))],
        out_specs=pl.BlockSpec((C, NBINS, NT), lambda i: (0, 0, i)),
        out_shape=jax.ShapeDtypeStruct((C, NBINS, N), jnp.bfloat16),
    )(pooled)


KSPLIT = 7
KC = NBINS * C // KSPLIT


def _fc_body(xt_ref, w6_ref, b6_ref, w7_ref, b7_ref, wc_ref, bc_ref,
             wb_ref, bb_ref, cls_ref, box_ref, acc_ref):
    k = pl.program_id(0)
    part = jax.lax.dot_general(
        w6_ref[...].astype(jnp.bfloat16), xt_ref[...],
        (((1,), (0,)), ((), ())), preferred_element_type=jnp.float32)

    @pl.when(k == 0)
    def _():
        acc_ref[...] = part

    @pl.when(k > 0)
    def _():
        acc_ref[...] += part

    @pl.when(k == KSPLIT - 1)
    def _():
        h1 = jnp.maximum(acc_ref[...] + jnp.transpose(b6_ref[...]), 0.0)
        h2 = jax.lax.dot_general(
            w7_ref[...], h1.astype(jnp.bfloat16), (((1,), (0,)), ((), ())),
            preferred_element_type=jnp.float32)
        h2 = jnp.maximum(h2 + jnp.transpose(b7_ref[...]), 0.0)
        cls_t = jax.lax.dot_general(
            wc_ref[...], h2.astype(jnp.bfloat16), (((1,), (0,)), ((), ())),
            preferred_element_type=jnp.float32)
        box_t = jax.lax.dot_general(
            wb_ref[...], h2.astype(jnp.bfloat16), (((1,), (0,)), ((), ())),
            preferred_element_type=jnp.float32)
        cls_ref[...] = jnp.transpose(cls_t) + bc_ref[...]
        box_ref[...] = jnp.transpose(box_t) + bb_ref[...]


def _fc_stack(xt, w6, b6, w7, b7, wc, bc, wb, bb):
    zero = lambda k: (0, 0)
    return pl.pallas_call(
        _fc_body,
        grid=(KSPLIT,),
        in_specs=[
            pl.BlockSpec((KC, N), lambda k: (k, 0)),
            pl.BlockSpec((FC, KC), lambda k: (0, k)),
            pl.BlockSpec((1, FC), zero),
            pl.BlockSpec((FC, FC), zero),
            pl.BlockSpec((1, FC), zero),
            pl.BlockSpec((NUM_CLASSES, FC), zero),
            pl.BlockSpec((1, NUM_CLASSES), zero),
            pl.BlockSpec((NUM_CLASSES * 4, FC), zero),
            pl.BlockSpec((1, NUM_CLASSES * 4), zero),
        ],
        out_specs=[
            pl.BlockSpec((N, NUM_CLASSES), zero),
            pl.BlockSpec((N, NUM_CLASSES * 4), zero),
        ],
        out_shape=[
            jax.ShapeDtypeStruct((N, NUM_CLASSES), jnp.float32),
            jax.ShapeDtypeStruct((N, NUM_CLASSES * 4), jnp.float32),
        ],
        scratch_shapes=[pltpu.VMEM((FC, N), jnp.float32)],
    )(xt, w6, b6, w7, b7, wc, bc, wb, bb)


def kernel(feat, proposals, img_shape, target, W6, b6, W7, b7, Wc, bc, Wb, bb):
    del img_shape, target
    table = _pack_table(feat[0].reshape(C, H * W))
    idx = _compute_indices(proposals).reshape(NBINS * 4, N)
    v = _gather_rows(table, idx)
    pooled = _combine(proposals, v.reshape(NBINS, 4, N, C // 2))
    xt = _xpose(pooled).reshape(NBINS * C, N)
    cls_scores, box_preds = _fc_stack(
        xt, W6, b6.reshape(1, FC),
        W7.astype(jnp.bfloat16), b7.reshape(1, FC),
        Wc.astype(jnp.bfloat16), bc.reshape(1, NUM_CLASSES),
        Wb.astype(jnp.bfloat16), bb.reshape(1, NUM_CLASSES * 4))
    return cls_scores, box_preds


# speedup vs baseline: 1.0660x; 1.0660x over previous
import functools

import jax
import jax.numpy as jnp
from jax.experimental import pallas as pl
from jax.experimental.pallas import tpu as pltpu
from jax.experimental.pallas import tpu_sc as plsc

C = 512
H = 50
W = 50
N = 512
POOL = 7
SCALE = 1.0 / 16.0
NUM_CLASSES = 21
FC = 1024
NBINS = POOL * POOL
NIDX = 4 * N * NBINS
KCH = POOL * C
GATHER_WIN = 128


def _bilinear_1d(lo, hi):
    lo = lo * SCALE
    hi = hi * SCALE
    ext = jnp.maximum(hi - lo, 1.0)
    g = (jax.lax.broadcasted_iota(jnp.int32, (1, POOL), 1).astype(jnp.float32)
         + 0.5) / POOL
    pos = lo + ext * g
    p0f = jnp.floor(pos)
    frac = pos - p0f
    return p0f, frac


def _corner_idx(p0f, limit):
    i0 = jnp.clip(p0f.astype(jnp.int32), 0, limit - 1)
    i1 = jnp.clip(i0 + 1, 0, limit - 1)
    return i0, i1


def _idx_body(prop_ref, idx_ref):
    p = prop_ref[...]
    x0f, _ = _bilinear_1d(p[:, 0:1], p[:, 2:3])
    y0f, _ = _bilinear_1d(p[:, 1:2], p[:, 3:4])
    x0, x1 = _corner_idx(x0f, W)
    y0, y1 = _corner_idx(y0f, H)
    x0t, x1t = jnp.transpose(x0), jnp.transpose(x1)
    y0t, y1t = jnp.transpose(y0), jnp.transpose(y1)
    for j, (yy, xx) in enumerate(((y0t, x0t), (y0t, x1t), (y1t, x0t), (y1t, x1t))):
        idx_ref[:, :, j, :] = yy[:, None, :] * W + xx[None, :, :]


def _compute_indices(proposals):
    return pl.pallas_call(
        _idx_body,
        out_shape=jax.ShapeDtypeStruct((POOL, POOL, 4, N), jnp.int32),
    )(proposals)


def _gather_rows(table, idx):
    mesh = plsc.VectorSubcoreMesh(core_axis_name="c", subcore_axis_name="s")
    D = table.shape[1]
    nidx = idx.shape[0] * idx.shape[1]

    @functools.partial(
        pl.kernel,
        out_type=jax.ShapeDtypeStruct((nidx, D), table.dtype),
        mesh=mesh,
    )
    def k(x_hbm, i_hbm, o_hbm):
        def body(i_vmem, o_vmem):
            pltpu.sync_copy(x_hbm.at[i_vmem.at[0]], o_vmem)

        blocks_per_row = idx.shape[1] // GATHER_WIN
        pltpu.emit_pipeline(
            body,
            grid=(nidx // GATHER_WIN,),
            in_specs=[pl.BlockSpec(
                (1, GATHER_WIN),
                lambda i: (i // blocks_per_row, i % blocks_per_row))],
            out_specs=[pl.BlockSpec((GATHER_WIN, D), lambda i: (i, 0))],
            core_axis_name=("c", "s"),
            dimension_semantics=(pltpu.PARALLEL,),
        )(i_hbm, o_hbm)

    return k(table, idx)


NB = 64


def _combine_body(prop_ref, v_ref, out_ref):
    p = prop_ref[...]
    _, fx = _bilinear_1d(p[:, 0:1], p[:, 2:3])
    _, fy = _bilinear_1d(p[:, 1:2], p[:, 3:4])
    hx, lx = 1.0 - fx, fx
    hy, ly = 1.0 - fy, fy
    himask = jnp.int32(-65536)
    for b in range(NBINS):
        py, px = b // POOL, b % POOL
        wy0 = hy[:, py:py + 1]
        wy1 = ly[:, py:py + 1]
        wx0 = hx[:, px:px + 1]
        wx1 = lx[:, px:px + 1]
        ws = (wy0 * wx0, wy0 * wx1, wy1 * wx0, wy1 * wx1)
        acc_lo = jnp.zeros((NB, C // 2), jnp.float32)
        acc_hi = jnp.zeros((NB, C // 2), jnp.float32)
        for j in range(4):
            slab = v_ref[b, j, :, :]
            lo = jax.lax.bitcast_convert_type(slab << 16, jnp.float32)
            hi = jax.lax.bitcast_convert_type(slab & himask, jnp.float32)
            acc_lo += ws[j] * lo
            acc_hi += ws[j] * hi
        out_ref[b, :, 0:C // 2] = acc_lo.astype(jnp.bfloat16)
        out_ref[b, :, C // 2:C] = acc_hi.astype(jnp.bfloat16)


def _combine(proposals, v4):
    return pl.pallas_call(
        _combine_body,
        grid=(N // NB,),
        in_specs=[
            pl.BlockSpec((NB, 4), lambda i: (i, 0)),
            pl.BlockSpec((NBINS, 4, NB, C // 2), lambda i: (0, 0, i, 0)),
        ],
        out_specs=pl.BlockSpec((NBINS, NB, C), lambda i: (0, i, 0)),
        out_shape=jax.ShapeDtypeStruct((NBINS, N, C), jnp.bfloat16),
    )(proposals, v4)


NT = 128


def _xpose_body(in_ref, out_ref):
    ii = jax.lax.broadcasted_iota(jnp.int32, (NT, NT), 0)
    jj = jax.lax.broadcasted_iota(jnp.int32, (NT, NT), 1)
    ident = (ii == jj).astype(jnp.bfloat16)
    for b in range(NBINS):
        t = jax.lax.dot_general(
            in_ref[b], ident, (((0,), (0,)), ((), ())),
            preferred_element_type=jnp.float32)
        out_ref[:, b, :] = t.astype(jnp.bfloat16)


def _xpose(pooled):
    return pl.pallas_call(
        _xpose_body,
        grid=(N // NT,),
        in_specs=[pl.BlockSpec((NBINS, NT, C), lambda i: (0, i, 0))],
        out_specs=pl.BlockSpec((C, NBINS, NT), lambda i: (0, 0, i)),
        out_shape=jax.ShapeDtypeStruct((C, NBINS, N), jnp.bfloat16),
    )(pooled)


KSPLIT = 7
KC = NBINS * C // KSPLIT


def _fc_body(xt_ref, w6_ref, b6_ref, w7_ref, b7_ref, wc_ref, bc_ref,
             wb_ref, bb_ref, cls_ref, box_ref, acc_ref):
    k = pl.program_id(0)
    part = jax.lax.dot_general(
        w6_ref[...].astype(jnp.bfloat16), xt_ref[...],
        (((1,), (0,)), ((), ())), preferred_element_type=jnp.float32)

    @pl.when(k == 0)
    def _():
        acc_ref[...] = part

    @pl.when(k > 0)
    def _():
        acc_ref[...] += part

    @pl.when(k == KSPLIT - 1)
    def _():
        h1 = jnp.maximum(acc_ref[...] + jnp.transpose(b6_ref[...]), 0.0)
        h2 = jax.lax.dot_general(
            w7_ref[...], h1.astype(jnp.bfloat16), (((1,), (0,)), ((), ())),
            preferred_element_type=jnp.float32)
        h2 = jnp.maximum(h2 + jnp.transpose(b7_ref[...]), 0.0)
        cls_t = jax.lax.dot_general(
            wc_ref[...], h2.astype(jnp.bfloat16), (((1,), (0,)), ((), ())),
            preferred_element_type=jnp.float32)
        box_t = jax.lax.dot_general(
            wb_ref[...], h2.astype(jnp.bfloat16), (((1,), (0,)), ((), ())),
            preferred_element_type=jnp.float32)
        cls_ref[...] = jnp.transpose(cls_t) + bc_ref[...]
        box_ref[...] = jnp.transpose(box_t) + bb_ref[...]


def _fc_stack(xt, w6, b6, w7, b7, wc, bc, wb, bb):
    zero = lambda k: (0, 0)
    return pl.pallas_call(
        _fc_body,
        grid=(KSPLIT,),
        in_specs=[
            pl.BlockSpec((KC, N), lambda k: (k, 0)),
            pl.BlockSpec((FC, KC), lambda k: (0, k)),
            pl.BlockSpec((1, FC), zero),
            pl.BlockSpec((FC, FC), zero),
            pl.BlockSpec((1, FC), zero),
            pl.BlockSpec((NUM_CLASSES, FC), zero),
            pl.BlockSpec((1, NUM_CLASSES), zero),
            pl.BlockSpec((NUM_CLASSES * 4, FC), zero),
            pl.BlockSpec((1, NUM_CLASSES * 4), zero),
        ],
        out_specs=[
            pl.BlockSpec((N, NUM_CLASSES), zero),
            pl.BlockSpec((N, NUM_CLASSES * 4), zero),
        ],
        out_shape=[
            jax.ShapeDtypeStruct((N, NUM_CLASSES), jnp.float32),
            jax.ShapeDtypeStruct((N, NUM_CLASSES * 4), jnp.float32),
        ],
        scratch_shapes=[pltpu.VMEM((FC, N), jnp.float32)],
    )(xt, w6, b6, w7, b7, wc, bc, wb, bb)


def kernel(feat, proposals, img_shape, target, W6, b6, W7, b7, Wc, bc, Wb, bb):
    del img_shape, target
    fb = feat[0].reshape(C, H * W).T.astype(jnp.bfloat16)
    table = jax.lax.bitcast_convert_type(
        jnp.stack([fb[:, :C // 2], fb[:, C // 2:]], axis=-1),
        jnp.int32)
    idx = _compute_indices(proposals).reshape(NBINS * 4, N)
    v = _gather_rows(table, idx)
    pooled = _combine(proposals, v.reshape(NBINS, 4, N, C // 2))
    xt = _xpose(pooled).reshape(NBINS * C, N)
    cls_scores, box_preds = _fc_stack(
        xt, W6, b6.reshape(1, FC),
        W7.astype(jnp.bfloat16), b7.reshape(1, FC),
        Wc.astype(jnp.bfloat16), bc.reshape(1, NUM_CLASSES),
        Wb.astype(jnp.bfloat16), bb.reshape(1, NUM_CLASSES * 4))
    return cls_scores, box_preds
